# Initial kernel scaffold; baseline (speedup 1.0000x reference)
import jax
import jax.numpy as jnp
from jax.experimental import pallas as pl


def kernel(attention_logits, gt_bboxes_3d, height_maps):
    raise NotImplementedError("write your pallas kernel here")


# speedup vs baseline: 28.8131x; 28.8131x over previous
import jax
import jax.numpy as jnp
from jax.experimental import pallas as pl
from jax.experimental.pallas import tpu as pltpu

_PC0, _PC1, _PC5 = -51.2, -51.2, 3.0
_GRID = 0.2
_POSW, _NEGW = 5.0, 0.1
_Y, _X = 512, 512
_B, _N = 4, 24
_RB = 64
_NR = _Y // _RB


def _loss_body(params_ref, x_ref, hm_ref, out_ref, gt_ref, acc_ref):
    b = pl.program_id(0)
    r = pl.program_id(1)
    row0 = (r * _RB).astype(jnp.float32)

    gt_ref[...] = jnp.zeros((_RB, _X), jnp.float32)
    rowf = jax.lax.broadcasted_iota(jnp.int32, (_RB, _X), 0).astype(jnp.float32) + row0
    colf = jax.lax.broadcasted_iota(jnp.int32, (_RB, _X), 1).astype(jnp.float32)

    for i in range(_N):
        cxg = params_ref[b, i, 0]
        cyg = params_ref[b, i, 1]
        cos_t = params_ref[b, i, 2]
        sin_t = params_ref[b, i, 3]
        hw = params_ref[b, i, 4]
        hl = params_ref[b, i, 5]
        hv = params_ref[b, i, 6]
        ymin = params_ref[b, i, 7]
        ymax = params_ref[b, i, 8]

        @pl.when(jnp.logical_and(ymax >= row0, ymin <= row0 + (_RB - 1)))
        def _():
            dx = colf - cxg
            dy = rowf - cyg
            l0 = dx * cos_t - dy * sin_t
            l1 = dx * sin_t + dy * cos_t
            inside = (jnp.abs(l0) <= hw) & (jnp.abs(l1) <= hl)
            gt_ref[...] = jnp.where(inside, hv, gt_ref[...])

    gt = gt_ref[...]
    x = x_ref[0, 0]
    hm = hm_ref[0, 0]

    pos = gt > 0.0
    weight = jnp.where(pos, _POSW, _NEGW)
    vf = (pos | (hm > 0.0)).astype(jnp.float32)

    bce = jnp.maximum(x, 0.0) - x * gt + jnp.log1p(jnp.exp(-jnp.abs(x)))
    p = jax.nn.sigmoid(x)
    p_t = p * gt + (1.0 - p) * (1.0 - gt)
    alpha_w = 0.25 * gt + 0.75 * (1.0 - gt)
    omp = 1.0 - p_t
    focal_w = omp * omp * alpha_w

    wb = weight * vf
    s_bce = jnp.sum(bce * wb)
    s_foc = jnp.sum(bce * focal_w * wb)
    s_cnt = jnp.sum(vf)

    @pl.when(r == 0)
    def _():
        acc_ref[b, 0] = 0.0
        acc_ref[b, 1] = 0.0
        acc_ref[b, 2] = 0.0

    acc_ref[b, 0] += s_bce
    acc_ref[b, 1] += s_foc
    acc_ref[b, 2] += s_cnt

    @pl.when(jnp.logical_and(b == _B - 1, r == _NR - 1))
    def _():
        total = jnp.float32(0.0)
        vs = jnp.float32(0.0)
        for bb in range(_B):
            cnt = acc_ref[bb, 2]
            denom = jnp.maximum(cnt, 1.0)
            comb = 0.5 * (acc_ref[bb, 0] + acc_ref[bb, 1]) / denom
            has_valid = (cnt > 0.0).astype(jnp.float32)
            total = total + comb * has_valid
            vs = vs + has_valid
        out_ref[0, 0] = jnp.where(vs > 0.0, total / jnp.maximum(vs, 1.0), total)


def _box_params(gt_bboxes_3d):
    cxg = (gt_bboxes_3d[..., 0] - _PC0) / _GRID
    cyg = (gt_bboxes_3d[..., 1] - _PC1) / _GRID
    wg2 = (gt_bboxes_3d[..., 3] / _GRID) / 2.0
    lg2 = (gt_bboxes_3d[..., 4] / _GRID) / 2.0
    theta = gt_bboxes_3d[..., 6]
    cos_t = jnp.cos(-theta)
    sin_t = jnp.sin(-theta)
    hv = gt_bboxes_3d[..., 5] / (_PC5 + 2.0)
    ey = jnp.abs(sin_t) * wg2 + jnp.abs(cos_t) * lg2
    return jnp.stack(
        [cxg, cyg, cos_t, sin_t, wg2, lg2, hv, cyg - ey, cyg + ey], axis=-1
    )


def kernel(attention_logits, gt_bboxes_3d, height_maps):
    params = _box_params(gt_bboxes_3d)
    out = pl.pallas_call(
        _loss_body,
        grid=(_B, _NR),
        in_specs=[
            pl.BlockSpec(memory_space=pltpu.SMEM),
            pl.BlockSpec((1, 1, _RB, _X), lambda b, r: (b, 0, r, 0)),
            pl.BlockSpec((1, 1, _RB, _X), lambda b, r: (b, 0, r, 0)),
        ],
        out_specs=pl.BlockSpec(memory_space=pltpu.SMEM),
        out_shape=jax.ShapeDtypeStruct((1, 1), jnp.float32),
        scratch_shapes=[
            pltpu.VMEM((_RB, _X), jnp.float32),
            pltpu.SMEM((_B, 3), jnp.float32),
        ],
    )(params, attention_logits, height_maps)
    return out[0, 0]
